# baseline (speedup 1.0000x reference)
import functools

import jax
import jax.numpy as jnp
from jax import lax
from jax.experimental import pallas as pl
from jax.experimental.pallas import tpu as pltpu
from jax.experimental.pallas import tpu_sc as plsc

N_CLASSES = 1000
BATCH = 16384
_L = 16

_NC = 2
_NS = 16
_NW = _NC * _NS
_ROWS_PER_W = BATCH // _NW
_C = 16
_NCHUNK = _ROWS_PER_W // _C
_BUF = _C * N_CLASSES
_NBUF = 4


def _onehot_body(p_hbm, cm_hbm, out_hbm, p_v, b0, b1, b2, b3, s0, s1, s2, s3):
    del cm_hbm
    bufs = (b0, b1, b2, b3)
    sems = (s0, s1, s2, s3)
    wid = lax.axis_index("s") * _NC + lax.axis_index("c")
    base = wid * _ROWS_PER_W
    pltpu.sync_copy(p_hbm.at[pl.ds(base, _ROWS_PER_W)], p_v)

    zeros16 = jnp.zeros((_L,), jnp.float32)
    ones16 = jnp.ones((_L,), jnp.float32)
    row_off = lax.iota(jnp.int32, _L) * N_CLASSES

    def zero(buf):
        def body(i, carry):
            for u in range(8):
                buf[pl.ds((i * 8 + u) * _L, _L)] = zeros16
            return carry
        lax.fori_loop(0, _BUF // (_L * 8), body, 0)

    for b in range(_NBUF):
        zero(bufs[b])

    def fire(b, off):
        idx = row_off + p_v[pl.ds(off, _L)]
        plsc.store_scatter(bufs[b], [idx], ones16)
        pltpu.async_copy(
            bufs[b], out_hbm.at[pl.ds((base + off) * N_CLASSES, _BUF)], sems[b])

    def drain(b):
        pltpu.make_async_copy(bufs[b], out_hbm.at[pl.ds(0, _BUF)], sems[b]).wait()

    for b in range(_NBUF):
        fire(b, b * _C)

    def group(gg, carry):
        off0 = gg * _NBUF * _C
        for b in range(_NBUF):
            off = off0 + b * _C
            drain(b)
            old_idx = row_off + p_v[pl.ds(off - _NBUF * _C, _L)]
            plsc.store_scatter(bufs[b], [old_idx], zeros16)
            fire(b, off)
        return carry

    lax.fori_loop(1, _NCHUNK // _NBUF, group, 0)
    for b in range(_NBUF):
        drain(b)


def kernel(p, class_matrix):
    mesh = plsc.VectorSubcoreMesh(core_axis_name="c", subcore_axis_name="s")
    run = functools.partial(
        pl.kernel,
        mesh=mesh,
        out_type=jax.ShapeDtypeStruct((BATCH * N_CLASSES,), jnp.float32),
        scratch_types=[
            pltpu.VMEM((_ROWS_PER_W,), jnp.int32),
        ] + [pltpu.VMEM((_BUF,), jnp.float32)] * _NBUF
          + [pltpu.SemaphoreType.DMA] * _NBUF,
        compiler_params=pltpu.CompilerParams(needs_layout_passes=False),
    )(_onehot_body)
    out = run(p.astype(jnp.int32), class_matrix)
    return out.reshape(BATCH, N_CLASSES)


# speedup vs baseline: 1.6081x; 1.6081x over previous
import functools

import jax
import jax.numpy as jnp
from jax import lax
from jax.experimental import pallas as pl
from jax.experimental.pallas import tpu as pltpu
from jax.experimental.pallas import tpu_sc as plsc

N_CLASSES = 1000
BATCH = 16384
_L = 16

_NC = 2
_NS = 16
_NW = _NC * _NS
_ROWS_PER_W = BATCH // _NW
_C = 16
_NCHUNK = _ROWS_PER_W // _C
_NBUF = 4


def _onehot_body(p_hbm, cm_hbm, out_hbm, p_v, b0, b1, b2, b3, s0, s1, s2, s3):
    del cm_hbm
    bufs = (b0, b1, b2, b3)
    sems = (s0, s1, s2, s3)
    wid = lax.axis_index("s") * _NC + lax.axis_index("c")
    base = wid * _ROWS_PER_W
    pltpu.sync_copy(p_hbm.at[pl.ds(base, _ROWS_PER_W)], p_v)

    zeros16 = jnp.zeros((_L,), jnp.float32)
    ones16 = jnp.ones((_L,), jnp.float32)
    rows16 = lax.iota(jnp.int32, _L)

    def zero(buf):
        def body(i, carry):
            for u in range(N_CLASSES // _L):
                buf[i, pl.ds(u * _L, _L)] = zeros16
            buf[i, pl.ds(N_CLASSES - _L, _L)] = zeros16
            return carry
        lax.fori_loop(0, _C, body, 0)

    for b in range(_NBUF):
        zero(bufs[b])

    def fire(b, off):
        cols = p_v[pl.ds(off, _L)]
        plsc.store_scatter(bufs[b], [rows16, cols], ones16)
        pltpu.async_copy(bufs[b], out_hbm.at[pl.ds(base + off, _C)], sems[b])

    def drain(b):
        pltpu.make_async_copy(bufs[b], out_hbm.at[pl.ds(0, _C)], sems[b]).wait()

    for b in range(_NBUF):
        fire(b, b * _C)

    def group(gg, carry):
        off0 = gg * _NBUF * _C
        for b in range(_NBUF):
            off = off0 + b * _C
            drain(b)
            old_cols = p_v[pl.ds(off - _NBUF * _C, _L)]
            plsc.store_scatter(bufs[b], [rows16, old_cols], zeros16)
            fire(b, off)
        return carry

    lax.fori_loop(1, _NCHUNK // _NBUF, group, 0)
    for b in range(_NBUF):
        drain(b)


def kernel(p, class_matrix):
    mesh = plsc.VectorSubcoreMesh(core_axis_name="c", subcore_axis_name="s")
    run = functools.partial(
        pl.kernel,
        mesh=mesh,
        out_type=jax.ShapeDtypeStruct((BATCH, N_CLASSES), jnp.float32),
        scratch_types=[
            pltpu.VMEM((_ROWS_PER_W,), jnp.int32),
        ] + [pltpu.VMEM((_C, N_CLASSES), jnp.float32)] * _NBUF
          + [pltpu.SemaphoreType.DMA] * _NBUF,
        compiler_params=pltpu.CompilerParams(needs_layout_passes=False),
    )(_onehot_body)
    return run(p.astype(jnp.int32), class_matrix)
